# baseline (speedup 1.0000x reference)
import functools

import jax
import jax.numpy as jnp
from jax import lax
from jax.experimental import pallas as pl
from jax.experimental.pallas import tpu as pltpu
from jax.experimental.pallas import tpu_sc as plsc

N = 10000
E = 320000
D = 128
H = 8
DH = 16

NC = 2
NS = 16
NW = NC * NS
EPW = E // NW
CH = 80
NCHUNK = EPW // CH
NP = 10240
RPT = NP // NS

_mesh = plsc.VectorSubcoreMesh(core_axis_name="c", subcore_axis_name="s")
_sc_params = pltpu.CompilerParams(
    use_tc_tiling_on_sc=False, needs_layout_passes=False)

_BCAST_DNUMS = lax.GatherDimensionNumbers(
    offset_dims=(), collapsed_slice_dims=(0,), start_index_map=(0,))


def _bcast_lane(v, lane):
    idx = jnp.full((16, 1), lane, jnp.int32)
    return lax.gather(v, idx, _BCAST_DNUMS, (1,),
                      mode=lax.GatherScatterMode.PROMISE_IN_BOUNDS)


def _perm_xor(v, idx_xor):
    return lax.gather(v, idx_xor, _BCAST_DNUMS, (1,),
                      mode=lax.GatherScatterMode.PROMISE_IN_BOUNDS)


def _qkv_body(x_ref, wq_ref, bq_ref, wk_ref, bk_ref, wv_ref, bv_ref,
              q_ref, k_ref, v_ref):
    x = x_ref[...]
    q_ref[...] = jnp.dot(x, wq_ref[...], preferred_element_type=jnp.float32) + bq_ref[...]
    k_ref[...] = jnp.dot(x, wk_ref[...], preferred_element_type=jnp.float32) + bk_ref[...]
    v_ref[...] = jnp.dot(x, wv_ref[...], preferred_element_type=jnp.float32) + bv_ref[...]


def _qkv(x, Wq, bq, Wk, bk, Wv, bv):
    BN = 400
    grid = (N // BN,)
    row = pl.BlockSpec((BN, D), lambda i: (i, 0))
    full = pl.BlockSpec((D, D), lambda i: (0, 0))
    bias = pl.BlockSpec((1, D), lambda i: (0, 0))
    return pl.pallas_call(
        _qkv_body,
        grid=grid,
        in_specs=[row, full, bias, full, bias, full, bias],
        out_specs=[row, row, row],
        out_shape=[jax.ShapeDtypeStruct((N, D), jnp.float32)] * 3,
    )(x, Wq, bq.reshape(1, D), Wk, bk.reshape(1, D), Wv, bv.reshape(1, D))


def _proj_body(p0_ref, p1_ref, wo_ref, bo_ref, o_ref):
    acc = p0_ref[...] + p1_ref[...]
    o_ref[...] = jnp.dot(acc, wo_ref[...], preferred_element_type=jnp.float32) + bo_ref[...]


def _out_proj(p0, p1, Wo, bo):
    BN = 400
    grid = (N // BN,)
    row = pl.BlockSpec((BN, D), lambda i: (i, 0))
    full = pl.BlockSpec((D, D), lambda i: (0, 0))
    bias = pl.BlockSpec((1, D), lambda i: (0, 0))
    return pl.pallas_call(
        _proj_body,
        grid=grid,
        in_specs=[row, row, full, bias],
        out_specs=row,
        out_shape=jax.ShapeDtypeStruct((N, D), jnp.float32),
    )(p0, p1, Wo, bo.reshape(1, D))


@functools.partial(
    pl.kernel,
    out_type=[
        jax.ShapeDtypeStruct((E, 16), jnp.float32),
        jax.ShapeDtypeStruct((NP, 16), jnp.float32),
        jax.ShapeDtypeStruct((NP, 16), jnp.float32),
    ],
    mesh=_mesh,
    compiler_params=_sc_params,
    scratch_types=[
        pltpu.VMEM((EPW,), jnp.int32),
        pltpu.VMEM((EPW,), jnp.int32),
        pltpu.VMEM((CH, D), jnp.float32),
        pltpu.VMEM((CH, D), jnp.float32),
        pltpu.VMEM((CH, D), jnp.float32),
        pltpu.VMEM((CH, D), jnp.float32),
        pltpu.VMEM((CH, 16), jnp.float32),
        pltpu.VMEM((CH, 16), jnp.float32),
        pltpu.VMEM((CH,), jnp.int32),
        pltpu.VMEM((CH,), jnp.int32),
        pltpu.VMEM_SHARED((NP, 16), jnp.float32),
        pltpu.SemaphoreType.DMA, pltpu.SemaphoreType.DMA,
        pltpu.SemaphoreType.DMA, pltpu.SemaphoreType.DMA,
        pltpu.SemaphoreType.DMA, pltpu.SemaphoreType.DMA,
        pltpu.SemaphoreType.DMA, pltpu.SemaphoreType.DMA,
    ],
)
def _sc_pass1(q_hbm, k_hbm, src_hbm, dst_hbm, z16_hbm,
              exw_hbm, d0_hbm, d1_hbm,
              sall, dall, qb0, qb1, kb0, kb1, ew0, ew1, sc0, sc1, denom_sh,
              sq0, sq1, sk0, sk1, se0, se1, sa0, sa1):
    c_ax = lax.axis_index("c")
    s_ax = lax.axis_index("s")
    wid = s_ax * NC + c_ax
    base = wid * EPW
    r0 = s_ax * RPT
    QB, KB, EW, SCI = (qb0, qb1), (kb0, kb1), (ew0, ew1), (sc0, sc1)
    SQ, SK, SE, SA = (sq0, sq1), (sk0, sk1), (se0, se1), (sa0, sa1)

    pltpu.sync_copy(z16_hbm.at[pl.ds(0, RPT)], denom_sh.at[pl.ds(r0, RPT)])
    pltpu.sync_copy(src_hbm.at[pl.ds(base, EPW)], sall)
    pltpu.sync_copy(dst_hbm.at[pl.ds(base, EPW)], dall)
    plsc.subcore_barrier()

    iota = lax.iota(jnp.int32, 16)

    def _issue_gather(c, b):
        qi = sall.at[pl.ds(c * CH, CH)]
        ki = dall.at[pl.ds(c * CH, CH)]
        pltpu.make_async_copy(q_hbm.at[qi], QB[b], SQ[b]).start()
        pltpu.make_async_copy(k_hbm.at[ki], KB[b], SK[b]).start()

    def _wait_gather(c, b):
        qi = sall.at[pl.ds(c * CH, CH)]
        ki = dall.at[pl.ds(c * CH, CH)]
        pltpu.make_async_copy(q_hbm.at[qi], QB[b], SQ[b]).wait()
        pltpu.make_async_copy(k_hbm.at[ki], KB[b], SK[b]).wait()

    def _wait_out(b):
        pltpu.make_async_copy(EW[b], exw_hbm.at[pl.ds(base, CH)], SE[b]).wait()
        pltpu.make_async_copy(EW[b], denom_sh.at[SCI[b]], SA[b]).wait()

    def _compute(b):
        neginf = jnp.where(iota < H, 0.0, -jnp.inf)
        masks = {m: (iota & m) != 0 for m in (1, 2, 4)}
        xors = {m: (iota ^ m)[:, None] for m in (1, 2, 4, 8)}

        def _merge(a, bb, m):
            c = jnp.where(masks[m], bb, a)
            d = jnp.where(masks[m], a, bb)
            return c + _perm_xor(d, xors[m])

        def _tree_edge(r):
            p = []
            for h in range(H):
                qv = QB[b][r, pl.ds(h * DH, DH)]
                kv = KB[b][r, pl.ds(h * DH, DH)]
                p.append(qv * kv)
            r01 = _merge(p[0], p[1], 1)
            r23 = _merge(p[2], p[3], 1)
            r45 = _merge(p[4], p[5], 1)
            r67 = _merge(p[6], p[7], 1)
            s03 = _merge(r01, r23, 2)
            s47 = _merge(r45, r67, 2)
            t = _merge(s03, s47, 4)
            return t + _perm_xor(t, xors[8]) + neginf

        UNR = 4

        def edge_body(g, _):
            r0 = g * UNR
            fs = [_tree_edge(r0 + u) for u in range(UNR)]
            es = [jnp.exp(f * 0.25) for f in fs]
            for u in range(UNR):
                EW[b][r0 + u, :] = es[u]
            return 0
        lax.fori_loop(0, CH // UNR, edge_body, 0)

    def _stage(c, b, issue_next, guard_wait):
        nb = 1 - b
        if issue_next:
            _issue_gather(c + 1, nb)
        _wait_gather(c, b)
        if guard_wait:
            pl.when(c >= 2)(lambda: _wait_out(b))
        else:
            _wait_out(b)
        _compute(b)
        for i in range(CH // 16):
            SCI[b][pl.ds(i * 16, 16)] = sall[pl.ds(c * CH + i * 16, 16)]
        e0 = base + c * CH
        pltpu.make_async_copy(EW[b], exw_hbm.at[pl.ds(e0, CH)], SE[b]).start()
        pltpu.make_async_copy(EW[b], denom_sh.at[SCI[b]], SA[b]).start(add=True)

    _issue_gather(0, 0)

    def outer(o, _):
        _stage(2 * o, 0, True, True)
        _stage(2 * o + 1, 1, True, True)
        return 0

    lax.fori_loop(0, NCHUNK // 2, outer, 0)
    _stage(NCHUNK - 1, 0, False, False)
    _wait_out(1)
    _wait_out(0)
    plsc.subcore_barrier()

    @pl.when(c_ax == 0)
    def _():
        pltpu.sync_copy(denom_sh.at[pl.ds(r0, RPT)], d0_hbm.at[pl.ds(r0, RPT)])

    @pl.when(c_ax == 1)
    def _():
        pltpu.sync_copy(denom_sh.at[pl.ds(r0, RPT)], d1_hbm.at[pl.ds(r0, RPT)])


@functools.partial(
    pl.kernel,
    out_type=[
        jax.ShapeDtypeStruct((NP, D), jnp.float32),
        jax.ShapeDtypeStruct((NP, D), jnp.float32),
    ],
    mesh=_mesh,
    compiler_params=_sc_params,
    scratch_types=[
        pltpu.VMEM((EPW,), jnp.int32),
        pltpu.VMEM((CH, D), jnp.float32),
        pltpu.VMEM((CH, D), jnp.float32),
        pltpu.VMEM((CH, 16), jnp.float32),
        pltpu.VMEM((CH, 16), jnp.float32),
        pltpu.VMEM((CH, 16), jnp.float32),
        pltpu.VMEM((CH, 16), jnp.float32),
        pltpu.VMEM((CH, 16), jnp.float32),
        pltpu.VMEM((CH, 16), jnp.float32),
        pltpu.VMEM((CH,), jnp.int32),
        pltpu.VMEM((CH,), jnp.int32),
        pltpu.VMEM_SHARED((NP, D), jnp.float32),
        pltpu.SemaphoreType.DMA, pltpu.SemaphoreType.DMA,
        pltpu.SemaphoreType.DMA, pltpu.SemaphoreType.DMA,
        pltpu.SemaphoreType.DMA, pltpu.SemaphoreType.DMA,
        pltpu.SemaphoreType.DMA, pltpu.SemaphoreType.DMA,
        pltpu.SemaphoreType.DMA, pltpu.SemaphoreType.DMA,
        pltpu.SemaphoreType.DMA, pltpu.SemaphoreType.DMA,
    ],
)
def _sc_pass2(v_hbm, exw_hbm, d0_hbm, d1_hbm, src_hbm, dst_hbm, z128_hbm,
              o0_hbm, o1_hbm,
              sall, vb0, vb1, xb0, xb1, da0, da1, db0, db1,
              sd0, sd1, out_sh,
              sv0, sv1, s00, s01, s10, s11, sx0, sx1, sdi0, sdi1, so0, so1):
    c_ax = lax.axis_index("c")
    s_ax = lax.axis_index("s")
    wid = s_ax * NC + c_ax
    base = wid * EPW
    r0 = s_ax * RPT
    VB, XB, DA, DB, SDI = (vb0, vb1), (xb0, xb1), (da0, da1), (db0, db1), (sd0, sd1)
    SV, S0, S1, SX = (sv0, sv1), (s00, s01), (s10, s11), (sx0, sx1)
    SD, SO = (sdi0, sdi1), (so0, so1)

    pltpu.sync_copy(z128_hbm, out_sh.at[pl.ds(r0, RPT)])
    pltpu.sync_copy(src_hbm.at[pl.ds(base, EPW)], sall)
    plsc.subcore_barrier()

    def _issue_in(c, b):
        si = sall.at[pl.ds(c * CH, CH)]
        pltpu.make_async_copy(v_hbm.at[si], VB[b], SV[b]).start()
        pltpu.make_async_copy(d0_hbm.at[si], DA[b], S0[b]).start()
        pltpu.make_async_copy(d1_hbm.at[si], DB[b], S1[b]).start()
        pltpu.make_async_copy(
            exw_hbm.at[pl.ds(base + c * CH, CH)], XB[b], SX[b]).start()
        pltpu.make_async_copy(
            dst_hbm.at[pl.ds(base + c * CH, CH)], SDI[b], SD[b]).start()

    def _wait_in(c, b):
        si = sall.at[pl.ds(c * CH, CH)]
        pltpu.make_async_copy(v_hbm.at[si], VB[b], SV[b]).wait()
        pltpu.make_async_copy(d0_hbm.at[si], DA[b], S0[b]).wait()
        pltpu.make_async_copy(d1_hbm.at[si], DB[b], S1[b]).wait()
        pltpu.make_async_copy(
            exw_hbm.at[pl.ds(base, CH)], XB[b], SX[b]).wait()

    def _wait_didx(b):
        pltpu.make_async_copy(
            dst_hbm.at[pl.ds(base, CH)], SDI[b], SD[b]).wait()

    def _wait_out(b):
        pltpu.make_async_copy(VB[b], out_sh.at[SDI[b]], SO[b]).wait()

    def _compute(b):
        UNR = 8

        def edge_body(g, _):
            r0 = g * UNR
            ws = []
            for u in range(UNR):
                r = r0 + u
                den = DA[b][r, :] + DB[b][r, :]
                ws.append(XB[b][r, :] / den)
            for u in range(0, UNR, 2):
                prods = []
                for uu in (u, u + 1):
                    r = r0 + uu
                    for h in range(H):
                        prods.append(VB[b][r, pl.ds(h * DH, DH)]
                                     * _bcast_lane(ws[uu], h))
                for i, uu in enumerate((u, u + 1)):
                    r = r0 + uu
                    for h in range(H):
                        VB[b][r, pl.ds(h * DH, DH)] = prods[i * H + h]
            return 0
        lax.fori_loop(0, CH // UNR, edge_body, 0)

    def _stage(c, b, issue_next, guard_prev):
        nb = 1 - b
        if guard_prev:
            pl.when(c >= 1)(lambda: _wait_out(nb))
        else:
            _wait_out(nb)
        if issue_next:
            _issue_in(c + 1, nb)
        _wait_in(c, b)
        _compute(b)
        _wait_didx(b)
        pltpu.make_async_copy(VB[b], out_sh.at[SDI[b]], SO[b]).start(add=True)

    _issue_in(0, 0)

    def outer(o, _):
        _stage(2 * o, 0, True, True)
        _stage(2 * o + 1, 1, True, True)
        return 0

    lax.fori_loop(0, NCHUNK // 2, outer, 0)
    _stage(NCHUNK - 1, 0, False, False)
    _wait_out(0)
    plsc.subcore_barrier()

    @pl.when(c_ax == 0)
    def _():
        pltpu.sync_copy(out_sh.at[pl.ds(r0, RPT)], o0_hbm.at[pl.ds(r0, RPT)])

    @pl.when(c_ax == 1)
    def _():
        pltpu.sync_copy(out_sh.at[pl.ds(r0, RPT)], o1_hbm.at[pl.ds(r0, RPT)])


def kernel(node_features, edge_index, Wq, bq, Wk, bk, Wv, bv, Wo, bo):
    src = edge_index[0]
    dst = edge_index[1]
    q, k, v = _qkv(node_features, Wq, bq, Wk, bk, Wv, bv)
    z16 = jnp.zeros((RPT, 16), jnp.float32)
    z128 = jnp.zeros((RPT, D), jnp.float32)
    exw, d0, d1 = _sc_pass1(q, k, src, dst, z16)
    o0, o1 = _sc_pass2(v, exw, d0, d1, src, dst, z128)
    return _out_proj(o0, o1, Wo, bo)


# speedup vs baseline: 1.0074x; 1.0074x over previous
import functools

import jax
import jax.numpy as jnp
from jax import lax
from jax.experimental import pallas as pl
from jax.experimental.pallas import tpu as pltpu
from jax.experimental.pallas import tpu_sc as plsc

N = 10000
E = 320000
D = 128
H = 8
DH = 16

NC = 2
NS = 16
NW = NC * NS
EPW = E // NW
CH = 80
NCHUNK = EPW // CH
NP = 10240
RPT = NP // NS

_mesh = plsc.VectorSubcoreMesh(core_axis_name="c", subcore_axis_name="s")
_sc_params = pltpu.CompilerParams(
    use_tc_tiling_on_sc=False, needs_layout_passes=False)

_BCAST_DNUMS = lax.GatherDimensionNumbers(
    offset_dims=(), collapsed_slice_dims=(0,), start_index_map=(0,))


def _bcast_lane(v, lane):
    idx = jnp.full((16, 1), lane, jnp.int32)
    return lax.gather(v, idx, _BCAST_DNUMS, (1,),
                      mode=lax.GatherScatterMode.PROMISE_IN_BOUNDS)


def _qkv_body(x_ref, wq_ref, bq_ref, wk_ref, bk_ref, wv_ref, bv_ref,
              q_ref, k_ref, v_ref):
    x = x_ref[...]
    q_ref[...] = jnp.dot(x, wq_ref[...], preferred_element_type=jnp.float32) + bq_ref[...]
    k_ref[...] = jnp.dot(x, wk_ref[...], preferred_element_type=jnp.float32) + bk_ref[...]
    v_ref[...] = jnp.dot(x, wv_ref[...], preferred_element_type=jnp.float32) + bv_ref[...]


def _qkv(x, Wq, bq, Wk, bk, Wv, bv):
    BN = 400
    grid = (N // BN,)
    row = pl.BlockSpec((BN, D), lambda i: (i, 0))
    full = pl.BlockSpec((D, D), lambda i: (0, 0))
    bias = pl.BlockSpec((1, D), lambda i: (0, 0))
    return pl.pallas_call(
        _qkv_body,
        grid=grid,
        in_specs=[row, full, bias, full, bias, full, bias],
        out_specs=[row, row, row],
        out_shape=[jax.ShapeDtypeStruct((N, D), jnp.float32)] * 3,
    )(x, Wq, bq.reshape(1, D), Wk, bk.reshape(1, D), Wv, bv.reshape(1, D))


def _proj_body(p0_ref, p1_ref, wo_ref, bo_ref, o_ref):
    acc = p0_ref[...] + p1_ref[...]
    o_ref[...] = jnp.dot(acc, wo_ref[...], preferred_element_type=jnp.float32) + bo_ref[...]


def _out_proj(p0, p1, Wo, bo):
    BN = 400
    grid = (N // BN,)
    row = pl.BlockSpec((BN, D), lambda i: (i, 0))
    full = pl.BlockSpec((D, D), lambda i: (0, 0))
    bias = pl.BlockSpec((1, D), lambda i: (0, 0))
    return pl.pallas_call(
        _proj_body,
        grid=grid,
        in_specs=[row, row, full, bias],
        out_specs=row,
        out_shape=jax.ShapeDtypeStruct((N, D), jnp.float32),
    )(p0, p1, Wo, bo.reshape(1, D))


@functools.partial(
    pl.kernel,
    out_type=[
        jax.ShapeDtypeStruct((E, 16), jnp.float32),
        jax.ShapeDtypeStruct((NP, 16), jnp.float32),
        jax.ShapeDtypeStruct((NP, 16), jnp.float32),
    ],
    mesh=_mesh,
    compiler_params=_sc_params,
    scratch_types=[
        pltpu.VMEM((EPW,), jnp.int32),
        pltpu.VMEM((EPW,), jnp.int32),
        pltpu.VMEM((CH, D), jnp.float32),
        pltpu.VMEM((CH, D), jnp.float32),
        pltpu.VMEM((CH, D), jnp.float32),
        pltpu.VMEM((CH, D), jnp.float32),
        pltpu.VMEM((CH, 16), jnp.float32),
        pltpu.VMEM((CH, 16), jnp.float32),
        pltpu.VMEM((CH,), jnp.int32),
        pltpu.VMEM((CH,), jnp.int32),
        pltpu.VMEM_SHARED((NP, 16), jnp.float32),
        pltpu.SemaphoreType.DMA, pltpu.SemaphoreType.DMA,
        pltpu.SemaphoreType.DMA, pltpu.SemaphoreType.DMA,
        pltpu.SemaphoreType.DMA, pltpu.SemaphoreType.DMA,
        pltpu.SemaphoreType.DMA, pltpu.SemaphoreType.DMA,
    ],
)
def _sc_pass1(q_hbm, k_hbm, src_hbm, dst_hbm, z16_hbm,
              exw_hbm, d0_hbm, d1_hbm,
              sall, dall, qb0, qb1, kb0, kb1, ew0, ew1, sc0, sc1, denom_sh,
              sq0, sq1, sk0, sk1, se0, se1, sa0, sa1):
    c_ax = lax.axis_index("c")
    s_ax = lax.axis_index("s")
    wid = s_ax * NC + c_ax
    base = wid * EPW
    r0 = s_ax * RPT
    QB, KB, EW, SCI = (qb0, qb1), (kb0, kb1), (ew0, ew1), (sc0, sc1)
    SQ, SK, SE, SA = (sq0, sq1), (sk0, sk1), (se0, se1), (sa0, sa1)

    pltpu.sync_copy(z16_hbm.at[pl.ds(0, RPT)], denom_sh.at[pl.ds(r0, RPT)])
    pltpu.sync_copy(src_hbm.at[pl.ds(base, EPW)], sall)
    pltpu.sync_copy(dst_hbm.at[pl.ds(base, EPW)], dall)
    plsc.subcore_barrier()

    iota = lax.iota(jnp.int32, 16)

    def _issue_gather(c, b):
        qi = sall.at[pl.ds(c * CH, CH)]
        ki = dall.at[pl.ds(c * CH, CH)]
        pltpu.make_async_copy(q_hbm.at[qi], QB[b], SQ[b]).start()
        pltpu.make_async_copy(k_hbm.at[ki], KB[b], SK[b]).start()

    def _wait_gather(c, b):
        qi = sall.at[pl.ds(c * CH, CH)]
        ki = dall.at[pl.ds(c * CH, CH)]
        pltpu.make_async_copy(q_hbm.at[qi], QB[b], SQ[b]).wait()
        pltpu.make_async_copy(k_hbm.at[ki], KB[b], SK[b]).wait()

    def _wait_out(b):
        pltpu.make_async_copy(EW[b], exw_hbm.at[pl.ds(base, CH)], SE[b]).wait()
        pltpu.make_async_copy(EW[b], denom_sh.at[SCI[b]], SA[b]).wait()

    def _compute(b):
        neginf = jnp.where(iota < H, 0.0, -jnp.inf)
        m1, m3, m5, m7 = (iota == 1), (iota == 3), (iota == 5), (iota == 7)
        ge2, ge6, ge4 = (iota >= 2), (iota >= 6), (iota >= 4)
        UNR = 4

        def _scan_edge(r):
            cums = []
            for h in range(H):
                qv = QB[b][r, pl.ds(h * DH, DH)]
                kv = KB[b][r, pl.ds(h * DH, DH)]
                cums.append(plsc.cumsum(qv * kv))
            return cums

        def _combine(cums):
            t = [_bcast_lane(c, 15) for c in cums]
            a01 = jnp.where(m1, t[1], t[0])
            a23 = jnp.where(m3, t[3], t[2])
            a45 = jnp.where(m5, t[5], t[4])
            a67 = jnp.where(m7, t[7], t[6])
            b03 = jnp.where(ge2, a23, a01)
            b47 = jnp.where(ge6, a67, a45)
            return jnp.where(ge4, b47, b03) + neginf

        def edge_body(g, _):
            r0 = g * UNR
            cs = [_scan_edge(r0 + u) for u in range(UNR)]
            es = [jnp.exp(_combine(c) * 0.25) for c in cs]
            for u in range(UNR):
                EW[b][r0 + u, :] = es[u]
            return 0
        lax.fori_loop(0, CH // UNR, edge_body, 0)

    def _stage(c, b, issue_next, guard_wait):
        nb = 1 - b
        if issue_next:
            _issue_gather(c + 1, nb)
        _wait_gather(c, b)
        if guard_wait:
            pl.when(c >= 2)(lambda: _wait_out(b))
        else:
            _wait_out(b)
        _compute(b)
        for i in range(CH // 16):
            SCI[b][pl.ds(i * 16, 16)] = sall[pl.ds(c * CH + i * 16, 16)]
        e0 = base + c * CH
        pltpu.make_async_copy(EW[b], exw_hbm.at[pl.ds(e0, CH)], SE[b]).start()
        pltpu.make_async_copy(EW[b], denom_sh.at[SCI[b]], SA[b]).start(add=True)

    _issue_gather(0, 0)

    def outer(o, _):
        _stage(2 * o, 0, True, True)
        _stage(2 * o + 1, 1, True, True)
        return 0

    lax.fori_loop(0, NCHUNK // 2, outer, 0)
    _stage(NCHUNK - 1, 0, False, False)
    _wait_out(1)
    _wait_out(0)
    plsc.subcore_barrier()

    @pl.when(c_ax == 0)
    def _():
        pltpu.sync_copy(denom_sh.at[pl.ds(r0, RPT)], d0_hbm.at[pl.ds(r0, RPT)])

    @pl.when(c_ax == 1)
    def _():
        pltpu.sync_copy(denom_sh.at[pl.ds(r0, RPT)], d1_hbm.at[pl.ds(r0, RPT)])


@functools.partial(
    pl.kernel,
    out_type=[
        jax.ShapeDtypeStruct((NP, D), jnp.float32),
        jax.ShapeDtypeStruct((NP, D), jnp.float32),
    ],
    mesh=_mesh,
    compiler_params=_sc_params,
    scratch_types=[
        pltpu.VMEM((EPW,), jnp.int32),
        pltpu.VMEM((CH, D), jnp.float32),
        pltpu.VMEM((CH, D), jnp.float32),
        pltpu.VMEM((CH, 16), jnp.float32),
        pltpu.VMEM((CH, 16), jnp.float32),
        pltpu.VMEM((CH, 16), jnp.float32),
        pltpu.VMEM((CH, 16), jnp.float32),
        pltpu.VMEM((CH, 16), jnp.float32),
        pltpu.VMEM((CH, 16), jnp.float32),
        pltpu.VMEM((CH,), jnp.int32),
        pltpu.VMEM((CH,), jnp.int32),
        pltpu.VMEM_SHARED((NP, D), jnp.float32),
        pltpu.SemaphoreType.DMA, pltpu.SemaphoreType.DMA,
        pltpu.SemaphoreType.DMA, pltpu.SemaphoreType.DMA,
        pltpu.SemaphoreType.DMA, pltpu.SemaphoreType.DMA,
        pltpu.SemaphoreType.DMA, pltpu.SemaphoreType.DMA,
        pltpu.SemaphoreType.DMA, pltpu.SemaphoreType.DMA,
        pltpu.SemaphoreType.DMA, pltpu.SemaphoreType.DMA,
    ],
)
def _sc_pass2(v_hbm, exw_hbm, d0_hbm, d1_hbm, src_hbm, dst_hbm, z128_hbm,
              o0_hbm, o1_hbm,
              sall, vb0, vb1, xb0, xb1, da0, da1, db0, db1,
              sd0, sd1, out_sh,
              sv0, sv1, s00, s01, s10, s11, sx0, sx1, sdi0, sdi1, so0, so1):
    c_ax = lax.axis_index("c")
    s_ax = lax.axis_index("s")
    wid = s_ax * NC + c_ax
    base = wid * EPW
    r0 = s_ax * RPT
    VB, XB, DA, DB, SDI = (vb0, vb1), (xb0, xb1), (da0, da1), (db0, db1), (sd0, sd1)
    SV, S0, S1, SX = (sv0, sv1), (s00, s01), (s10, s11), (sx0, sx1)
    SD, SO = (sdi0, sdi1), (so0, so1)

    pltpu.sync_copy(z128_hbm, out_sh.at[pl.ds(r0, RPT)])
    pltpu.sync_copy(src_hbm.at[pl.ds(base, EPW)], sall)
    plsc.subcore_barrier()

    def _issue_in(c, b):
        si = sall.at[pl.ds(c * CH, CH)]
        pltpu.make_async_copy(v_hbm.at[si], VB[b], SV[b]).start()
        pltpu.make_async_copy(d0_hbm.at[si], DA[b], S0[b]).start()
        pltpu.make_async_copy(d1_hbm.at[si], DB[b], S1[b]).start()
        pltpu.make_async_copy(
            exw_hbm.at[pl.ds(base + c * CH, CH)], XB[b], SX[b]).start()
        pltpu.make_async_copy(
            dst_hbm.at[pl.ds(base + c * CH, CH)], SDI[b], SD[b]).start()

    def _wait_in(c, b):
        si = sall.at[pl.ds(c * CH, CH)]
        pltpu.make_async_copy(v_hbm.at[si], VB[b], SV[b]).wait()
        pltpu.make_async_copy(d0_hbm.at[si], DA[b], S0[b]).wait()
        pltpu.make_async_copy(d1_hbm.at[si], DB[b], S1[b]).wait()
        pltpu.make_async_copy(
            exw_hbm.at[pl.ds(base, CH)], XB[b], SX[b]).wait()

    def _wait_didx(b):
        pltpu.make_async_copy(
            dst_hbm.at[pl.ds(base, CH)], SDI[b], SD[b]).wait()

    def _wait_out(b):
        pltpu.make_async_copy(VB[b], out_sh.at[SDI[b]], SO[b]).wait()

    def _compute(b):
        UNR = 8

        def edge_body(g, _):
            r0 = g * UNR
            ws = []
            for u in range(UNR):
                r = r0 + u
                den = DA[b][r, :] + DB[b][r, :]
                ws.append(XB[b][r, :] / den)
            for u in range(0, UNR, 2):
                prods = []
                for uu in (u, u + 1):
                    r = r0 + uu
                    for h in range(H):
                        prods.append(VB[b][r, pl.ds(h * DH, DH)]
                                     * _bcast_lane(ws[uu], h))
                for i, uu in enumerate((u, u + 1)):
                    r = r0 + uu
                    for h in range(H):
                        VB[b][r, pl.ds(h * DH, DH)] = prods[i * H + h]
            return 0
        lax.fori_loop(0, CH // UNR, edge_body, 0)

    def _stage(c, b, issue_next, guard_prev):
        nb = 1 - b
        if guard_prev:
            pl.when(c >= 1)(lambda: _wait_out(nb))
        else:
            _wait_out(nb)
        if issue_next:
            _issue_in(c + 1, nb)
        _wait_in(c, b)
        _compute(b)
        _wait_didx(b)
        pltpu.make_async_copy(VB[b], out_sh.at[SDI[b]], SO[b]).start(add=True)

    _issue_in(0, 0)

    def outer(o, _):
        _stage(2 * o, 0, True, True)
        _stage(2 * o + 1, 1, True, True)
        return 0

    lax.fori_loop(0, NCHUNK // 2, outer, 0)
    _stage(NCHUNK - 1, 0, False, False)
    _wait_out(0)
    plsc.subcore_barrier()

    @pl.when(c_ax == 0)
    def _():
        pltpu.sync_copy(out_sh.at[pl.ds(r0, RPT)], o0_hbm.at[pl.ds(r0, RPT)])

    @pl.when(c_ax == 1)
    def _():
        pltpu.sync_copy(out_sh.at[pl.ds(r0, RPT)], o1_hbm.at[pl.ds(r0, RPT)])


def kernel(node_features, edge_index, Wq, bq, Wk, bk, Wv, bv, Wo, bo):
    src = edge_index[0]
    dst = edge_index[1]
    q, k, v = _qkv(node_features, Wq, bq, Wk, bk, Wv, bv)
    z16 = jnp.zeros((RPT, 16), jnp.float32)
    z128 = jnp.zeros((RPT, D), jnp.float32)
    exw, d0, d1 = _sc_pass1(q, k, src, dst, z16)
    o0, o1 = _sc_pass2(v, exw, d0, d1, src, dst, z128)
    return _out_proj(o0, o1, Wo, bo)
